# baseline (speedup 1.0000x reference)
import functools

import jax
import jax.numpy as jnp
from jax import lax
from jax.experimental import pallas as pl
from jax.experimental.pallas import tpu as pltpu
from jax.experimental.pallas import tpu_sc as plsc

EMBEDDING_DIM = 64
_NUM_CORES = 2
_NUM_SUBCORES = 16
_NW = _NUM_CORES * _NUM_SUBCORES

_GBUF = 8
_SBUF = 4


def _build(batch: int, seq: int):
    assert batch // _NW == 128 and seq % _GBUF == 0
    assert EMBEDDING_DIM == 64
    n_eb = EMBEDDING_DIM // 8

    mesh = plsc.VectorSubcoreMesh(core_axis_name="c", subcore_axis_name="s")

    @functools.partial(
        pl.kernel,
        mesh=mesh,
        compiler_params=pltpu.CompilerParams(
            use_tc_tiling_on_sc=False, needs_layout_passes=False
        ),
        out_type=jax.ShapeDtypeStruct((seq, n_eb, _NW, 8, 128), jnp.float32),
        scratch_types=[
            pltpu.VMEM((seq, 128), jnp.int32),
            pltpu.VMEM((_GBUF, 128, EMBEDDING_DIM), jnp.float32),
            pltpu.VMEM((_SBUF, n_eb, 8, 129), jnp.float32),
            pltpu.SemaphoreType.DMA((_GBUF,)),
            pltpu.SemaphoreType.DMA((_SBUF,)),
        ],
    )
    def emb(idxt_hbm, table_hbm, out_hbm, idx_t, rows_v, tile_v, gsem, ssem):
        wid = lax.axis_index("s") * _NUM_CORES + lax.axis_index("c")

        pltpu.sync_copy(idxt_hbm.at[:, pl.ds(wid * 128, 128)], idx_t)

        def issue_gather(s, b):
            pltpu.async_copy(
                table_hbm.at[idx_t.at[s]], rows_v.at[b], gsem.at[b]
            )

        def wait_gather(s, b):
            pltpu.make_async_copy(
                table_hbm.at[idx_t.at[s]], rows_v.at[b], gsem.at[b]
            ).wait()

        def issue_store(s, tb):
            pltpu.async_copy(
                tile_v.at[tb, :, :, pl.ds(0, 128)],
                out_hbm.at[s, :, wid],
                ssem.at[tb],
            )

        def wait_store(tb):
            pltpu.make_async_copy(
                tile_v.at[tb, :, :, pl.ds(0, 128)],
                out_hbm.at[0, :, wid],
                ssem.at[tb],
            ).wait()

        for b in range(_GBUF):
            issue_gather(b, b)

        lanes = lax.iota(jnp.int32, 16)
        zeros16 = jnp.zeros((16,), jnp.int32)
        ce_big = [(16 * k + lanes) // 8 for k in range(4)]
        ce_small = [(16 * k + lanes) % 8 for k in range(4)]

        def group_body(g, carry):
            for u in range(_GBUF):
                s = g * _GBUF + u
                tb = u % _SBUF
                wait_gather(s, u)

                @pl.when(s >= _SBUF)
                def _():
                    wait_store(tb)

                @plsc.parallel_loop(0, 128, 1, unroll=16)
                def _(b):
                    idxb = zeros16 + b
                    for k in range(4):
                        v = rows_v[u, b, pl.ds(16 * k, 16)]
                        plsc.store_scatter(
                            tile_v.at[tb], [ce_big[k], ce_small[k], idxb], v
                        )

                issue_store(s, tb)

                @pl.when(s + _GBUF < seq)
                def _():
                    issue_gather(s + _GBUF, u)

            return carry

        lax.fori_loop(0, seq // _GBUF, group_body, 0)

        for tb in range(_SBUF):
            wait_store(tb)

    return emb


def kernel(input, weight):
    batch, seq = input.shape
    out5 = _build(batch, seq)(input.T.astype(jnp.int32), weight)
    return out5.transpose(2, 4, 0, 1, 3).reshape(batch, seq, EMBEDDING_DIM)


# speedup vs baseline: 1.0108x; 1.0108x over previous
import functools

import jax
import jax.numpy as jnp
from jax import lax
from jax.experimental import pallas as pl
from jax.experimental.pallas import tpu as pltpu
from jax.experimental.pallas import tpu_sc as plsc

EMBEDDING_DIM = 64
_NUM_CORES = 2
_NUM_SUBCORES = 16
_NW = _NUM_CORES * _NUM_SUBCORES

_GBUF = 4
_SBUF = 2


def _build(batch: int, seq: int):
    assert batch // _NW == 128 and seq % _GBUF == 0
    assert EMBEDDING_DIM == 64
    n_eb = EMBEDDING_DIM // 8

    mesh = plsc.VectorSubcoreMesh(core_axis_name="c", subcore_axis_name="s")

    @functools.partial(
        pl.kernel,
        mesh=mesh,
        compiler_params=pltpu.CompilerParams(
            use_tc_tiling_on_sc=False, needs_layout_passes=False
        ),
        out_type=jax.ShapeDtypeStruct((seq, n_eb, _NW, 8, 128), jnp.float32),
        scratch_types=[
            pltpu.VMEM((seq, 128), jnp.int32),
            pltpu.VMEM((_GBUF, 128, EMBEDDING_DIM), jnp.float32),
            pltpu.VMEM((_SBUF, n_eb, 8, 129), jnp.float32),
            pltpu.SemaphoreType.DMA((_GBUF,)),
            pltpu.SemaphoreType.DMA((_SBUF,)),
        ],
    )
    def emb(idxt_hbm, table_hbm, out_hbm, idx_t, rows_v, tile_v, gsem, ssem):
        wid = lax.axis_index("s") * _NUM_CORES + lax.axis_index("c")

        pltpu.sync_copy(idxt_hbm.at[:, pl.ds(wid * 128, 128)], idx_t)

        def issue_gather(s, b):
            pltpu.async_copy(
                table_hbm.at[idx_t.at[s]], rows_v.at[b], gsem.at[b]
            )

        def wait_gather(s, b):
            pltpu.make_async_copy(
                table_hbm.at[idx_t.at[s]], rows_v.at[b], gsem.at[b]
            ).wait()

        def issue_store(s, tb):
            pltpu.async_copy(
                tile_v.at[tb, :, :, pl.ds(0, 128)],
                out_hbm.at[s, :, wid],
                ssem.at[tb],
            )

        def wait_store(tb):
            pltpu.make_async_copy(
                tile_v.at[tb, :, :, pl.ds(0, 128)],
                out_hbm.at[0, :, wid],
                ssem.at[tb],
            ).wait()

        for b in range(_GBUF):
            issue_gather(b, b)

        lanes = lax.iota(jnp.int32, 16)
        zeros16 = jnp.zeros((16,), jnp.int32)
        ce_big = [(16 * k + lanes) // 8 for k in range(4)]
        ce_small = [(16 * k + lanes) % 8 for k in range(4)]

        def group_body(g, carry):
            for u in range(_GBUF):
                s = g * _GBUF + u
                tb = u % _SBUF
                wait_gather(s, u)

                @pl.when(s >= _SBUF)
                def _():
                    wait_store(tb)

                @plsc.parallel_loop(0, 128, 1, unroll=16)
                def _(b):
                    idxb = zeros16 + b
                    for k in range(4):
                        v = rows_v[u, b, pl.ds(16 * k, 16)]
                        plsc.store_scatter(
                            tile_v.at[tb], [ce_big[k], ce_small[k], idxb], v
                        )

                issue_store(s, tb)

                @pl.when(s + _GBUF < seq)
                def _():
                    issue_gather(s + _GBUF, u)

            return carry

        lax.fori_loop(0, seq // _GBUF, group_body, 0)

        for tb in range(_SBUF):
            wait_store(tb)

    return emb


def kernel(input, weight):
    batch, seq = input.shape
    out5 = _build(batch, seq)(input.T.astype(jnp.int32), weight)
    return out5.transpose(2, 4, 0, 1, 3).reshape(batch, seq, EMBEDDING_DIM)
